# baseline (speedup 1.0000x reference)
import functools

import jax
import jax.numpy as jnp
import numpy as np
from jax import lax
from jax.experimental import pallas as pl
from jax.experimental.pallas import tpu as pltpu
from jax.experimental.pallas import tpu_sc as plsc

SEQ_LEN = 200
VOCAB = 1000000
EMBED_DIM = 64
BATCH = 1024

NUM_CORES = 2
NUM_SUBCORES = 16
NUM_WORKERS = NUM_CORES * NUM_SUBCORES

TOTAL = BATCH * SEQ_LEN
ROWS_PER_WORKER = TOTAL // NUM_WORKERS
CHUNK = 128
CHUNKS_PER_WORKER = ROWS_PER_WORKER // CHUNK
IDX_ROWS = TOTAL // CHUNK


def _positional_encoding():
    pos = np.arange(SEQ_LEN, dtype=np.float64)[:, None]
    i = np.arange(EMBED_DIM, dtype=np.float64)[None, :]
    exponent = (i - np.mod(i, 2)) / EMBED_DIM
    angle = pos / np.power(10000.0, exponent)
    pe = np.where(np.mod(np.arange(EMBED_DIM), 2)[None, :] == 0,
                  np.sin(angle), np.cos(angle))
    return pe.astype(np.float32)


_PE = _positional_encoding()


def _sc_body(table_hbm, idx_hbm, pe_hbm, out_hbm, idx_v, pe_v, rows_v, gsem):
    wid = lax.axis_index("s") * NUM_CORES + lax.axis_index("c")
    row_base = wid * ROWS_PER_WORKER

    pltpu.sync_copy(idx_hbm.at[pl.ds(row_base, ROWS_PER_WORKER)], idx_v)
    pltpu.sync_copy(pe_hbm, pe_v)

    def chunk_body(j, _):
        idx_slice = idx_v.at[pl.ds(j * CHUNK, CHUNK)]
        pltpu.async_copy(table_hbm.at[idx_slice], rows_v, gsem).wait()
        p0 = lax.rem(j * CHUNK, SEQ_LEN)

        def row_body(r, p):
            for k in range(EMBED_DIM // 16):
                sl = pl.ds(16 * k, 16)
                rows_v[r, sl] = rows_v[r, sl] + pe_v[p, sl]
            return jnp.where(p == SEQ_LEN - 1, 0, p + 1)

        lax.fori_loop(0, CHUNK, row_body, p0)
        pltpu.sync_copy(rows_v, out_hbm.at[pl.ds(row_base + j * CHUNK, CHUNK)])
        return 0

    lax.fori_loop(0, CHUNKS_PER_WORKER, chunk_body, 0)


@jax.jit
def _embed(inputs_flat, table, pe):
    mesh = plsc.VectorSubcoreMesh(core_axis_name="c", subcore_axis_name="s")
    call = functools.partial(
        pl.kernel,
        mesh=mesh,
        out_type=jax.ShapeDtypeStruct((TOTAL, EMBED_DIM), jnp.float32),
        scratch_types=[
            pltpu.VMEM((ROWS_PER_WORKER,), jnp.int32),
            pltpu.VMEM((SEQ_LEN, EMBED_DIM), jnp.float32),
            pltpu.VMEM((CHUNK, EMBED_DIM), jnp.float32),
            pltpu.SemaphoreType.DMA,
        ],
        compiler_params=pltpu.CompilerParams(use_tc_tiling_on_sc=False),
    )(_sc_body)
    return call(table, inputs_flat, pe)


def kernel(inputs, table):
    idx = inputs.reshape(TOTAL).astype(jnp.int32)
    out = _embed(idx, table, _PE)
    return out.reshape(BATCH, SEQ_LEN, EMBED_DIM)


# speedup vs baseline: 1.1439x; 1.1439x over previous
import functools

import jax
import jax.numpy as jnp
import numpy as np
from jax import lax
from jax.experimental import pallas as pl
from jax.experimental.pallas import tpu as pltpu
from jax.experimental.pallas import tpu_sc as plsc

SEQ_LEN = 200
VOCAB = 1000000
EMBED_DIM = 64
BATCH = 1024

NUM_CORES = 2
NUM_SUBCORES = 16
NUM_WORKERS = NUM_CORES * NUM_SUBCORES

TOTAL = BATCH * SEQ_LEN
ROWS_PER_WORKER = TOTAL // NUM_WORKERS
CHUNK = 128
CHUNKS_PER_WORKER = ROWS_PER_WORKER // CHUNK
IDX_ROWS = TOTAL // CHUNK


def _positional_encoding():
    pos = np.arange(SEQ_LEN, dtype=np.float64)[:, None]
    i = np.arange(EMBED_DIM, dtype=np.float64)[None, :]
    exponent = (i - np.mod(i, 2)) / EMBED_DIM
    angle = pos / np.power(10000.0, exponent)
    pe = np.where(np.mod(np.arange(EMBED_DIM), 2)[None, :] == 0,
                  np.sin(angle), np.cos(angle))
    return pe.astype(np.float32)


_PE = _positional_encoding()


def _sc_body(table_hbm, idx_hbm, pe_hbm, out_hbm, idx_v, pe_s, rows_v, gsem):
    sid = lax.axis_index("s")
    wid = sid * NUM_CORES + lax.axis_index("c")
    row_base = wid * ROWS_PER_WORKER

    pltpu.sync_copy(idx_hbm.at[pl.ds(row_base, ROWS_PER_WORKER)], idx_v)

    @pl.when(sid == 0)
    def _():
        pltpu.sync_copy(pe_hbm, pe_s)

    plsc.subcore_barrier()

    def chunk_body(j, _):
        base = j * SEQ_LEN
        pltpu.sync_copy(pe_s, rows_v)
        cp1 = pltpu.async_copy(
            table_hbm.at[idx_v.at[pl.ds(base, 128)]],
            rows_v.at[pl.ds(0, 128)], gsem, add=True)
        cp2 = pltpu.async_copy(
            table_hbm.at[idx_v.at[pl.ds(base + 128, 72)]],
            rows_v.at[pl.ds(128, 72)], gsem, add=True)
        cp1.wait()
        cp2.wait()
        pltpu.sync_copy(rows_v, out_hbm.at[pl.ds(row_base + base, SEQ_LEN)])
        return 0

    lax.fori_loop(0, ROWS_PER_WORKER // SEQ_LEN, chunk_body, 0)


@jax.jit
def _embed(inputs_flat, table, pe):
    mesh = plsc.VectorSubcoreMesh(core_axis_name="c", subcore_axis_name="s")
    call = functools.partial(
        pl.kernel,
        mesh=mesh,
        out_type=jax.ShapeDtypeStruct((TOTAL, EMBED_DIM), jnp.float32),
        scratch_types=[
            pltpu.VMEM((ROWS_PER_WORKER,), jnp.int32),
            pltpu.VMEM_SHARED((SEQ_LEN, EMBED_DIM), jnp.float32),
            pltpu.VMEM((SEQ_LEN, EMBED_DIM), jnp.float32),
            pltpu.SemaphoreType.DMA,
        ],
        compiler_params=pltpu.CompilerParams(use_tc_tiling_on_sc=False),
    )(_sc_body)
    return call(table, inputs_flat, pe)


def kernel(inputs, table):
    idx = inputs.reshape(TOTAL).astype(jnp.int32)
    out = _embed(idx, table, _PE)
    return out.reshape(BATCH, SEQ_LEN, EMBED_DIM)


# speedup vs baseline: 1.1960x; 1.0456x over previous
import functools

import jax
import jax.numpy as jnp
import numpy as np
from jax import lax
from jax.experimental import pallas as pl
from jax.experimental.pallas import tpu as pltpu
from jax.experimental.pallas import tpu_sc as plsc

SEQ_LEN = 200
VOCAB = 1000000
EMBED_DIM = 64
BATCH = 1024

NUM_CORES = 2
NUM_SUBCORES = 16
NUM_WORKERS = NUM_CORES * NUM_SUBCORES

TOTAL = BATCH * SEQ_LEN
ROWS_PER_WORKER = TOTAL // NUM_WORKERS
CHUNK = 128
CHUNKS_PER_WORKER = ROWS_PER_WORKER // CHUNK
IDX_ROWS = TOTAL // CHUNK


def _positional_encoding():
    pos = np.arange(SEQ_LEN, dtype=np.float64)[:, None]
    i = np.arange(EMBED_DIM, dtype=np.float64)[None, :]
    exponent = (i - np.mod(i, 2)) / EMBED_DIM
    angle = pos / np.power(10000.0, exponent)
    pe = np.where(np.mod(np.arange(EMBED_DIM), 2)[None, :] == 0,
                  np.sin(angle), np.cos(angle))
    return pe.astype(np.float32)


_PE = _positional_encoding()


NCHUNK = ROWS_PER_WORKER // SEQ_LEN
NBUF = 8
LAG_G = 2
LAG_O = 4


def _sc_body(table_hbm, idx_hbm, pe_hbm, out_hbm, idx_v, pe_s, bufs, sems):
    sid = lax.axis_index("s")
    wid = sid * NUM_CORES + lax.axis_index("c")
    row_base = wid * ROWS_PER_WORKER

    pltpu.sync_copy(idx_hbm.at[pl.ds(row_base, ROWS_PER_WORKER)], idx_v)

    @pl.when(sid == 0)
    def _():
        pltpu.sync_copy(pe_hbm, pe_s)

    plsc.subcore_barrier()

    fills, g1s, g2s, outs = {}, {}, {}, {}
    for t in range(NCHUNK + LAG_O):
        if t < NCHUNK:
            b = t % NBUF
            if t >= NBUF:
                outs[t - NBUF].wait()
            fills[t] = pltpu.async_copy(pe_s, bufs.at[b], sems.at[b])
        j = t - LAG_G
        if 0 <= j < NCHUNK:
            b = j % NBUF
            fills[j].wait()
            base = j * SEQ_LEN
            g1s[j] = pltpu.async_copy(
                table_hbm.at[idx_v.at[pl.ds(base, 128)]],
                bufs.at[b].at[pl.ds(0, 128)], sems.at[b], add=True)
            g2s[j] = pltpu.async_copy(
                table_hbm.at[idx_v.at[pl.ds(base + 128, 72)]],
                bufs.at[b].at[pl.ds(128, 72)], sems.at[b], add=True)
        j = t - LAG_O
        if 0 <= j < NCHUNK:
            b = j % NBUF
            g1s[j].wait()
            g2s[j].wait()
            outs[j] = pltpu.async_copy(
                bufs.at[b], out_hbm.at[pl.ds(row_base + j * SEQ_LEN, SEQ_LEN)],
                sems.at[b])
    for j in range(NCHUNK - NBUF, NCHUNK):
        outs[j].wait()


@jax.jit
def _embed(inputs_flat, table, pe):
    mesh = plsc.VectorSubcoreMesh(core_axis_name="c", subcore_axis_name="s")
    call = functools.partial(
        pl.kernel,
        mesh=mesh,
        out_type=jax.ShapeDtypeStruct((TOTAL, EMBED_DIM), jnp.float32),
        scratch_types=[
            pltpu.VMEM((ROWS_PER_WORKER,), jnp.int32),
            pltpu.VMEM_SHARED((SEQ_LEN, EMBED_DIM), jnp.float32),
            pltpu.VMEM((NBUF, SEQ_LEN, EMBED_DIM), jnp.float32),
            pltpu.SemaphoreType.DMA((NBUF,)),
        ],
        compiler_params=pltpu.CompilerParams(use_tc_tiling_on_sc=False),
    )(_sc_body)
    return call(table, inputs_flat, pe)


def kernel(inputs, table):
    idx = inputs.reshape(TOTAL).astype(jnp.int32)
    out = _embed(idx, table, _PE)
    return out.reshape(BATCH, SEQ_LEN, EMBED_DIM)


# speedup vs baseline: 1.3786x; 1.1526x over previous
import functools

import jax
import jax.numpy as jnp
import numpy as np
from jax import lax
from jax.experimental import pallas as pl
from jax.experimental.pallas import tpu as pltpu
from jax.experimental.pallas import tpu_sc as plsc

SEQ_LEN = 200
VOCAB = 1000000
EMBED_DIM = 64
BATCH = 1024

NUM_CORES = 2
NUM_SUBCORES = 16
NUM_WORKERS = NUM_CORES * NUM_SUBCORES

TOTAL = BATCH * SEQ_LEN
ROWS_PER_WORKER = TOTAL // NUM_WORKERS
CHUNK = 128
CHUNKS_PER_WORKER = ROWS_PER_WORKER // CHUNK
IDX_ROWS = TOTAL // CHUNK


def _positional_encoding():
    pos = np.arange(SEQ_LEN, dtype=np.float64)[:, None]
    i = np.arange(EMBED_DIM, dtype=np.float64)[None, :]
    exponent = (i - np.mod(i, 2)) / EMBED_DIM
    angle = pos / np.power(10000.0, exponent)
    pe = np.where(np.mod(np.arange(EMBED_DIM), 2)[None, :] == 0,
                  np.sin(angle), np.cos(angle))
    return pe.astype(np.float32)


_PE = _positional_encoding()


NCHUNK = ROWS_PER_WORKER // SEQ_LEN
NBUF = 4
LAG_G = 1
LAG_O = 2
PAD_DIM = 2 * EMBED_DIM


def _sc_body(table_hbm, idx_hbm, pe_hbm, out_hbm, idx_v, pe_s, bufs, *sems):
    sid = lax.axis_index("s")
    wid = sid * NUM_CORES + lax.axis_index("c")
    row_base = wid * ROWS_PER_WORKER

    pltpu.sync_copy(idx_hbm.at[pl.ds(row_base, ROWS_PER_WORKER)], idx_v)

    @pl.when(sid == 0)
    def _():
        pltpu.sync_copy(pe_hbm, pe_s)

    plsc.subcore_barrier()

    fills, g1s, g2s, outs = {}, {}, {}, {}
    for t in range(NCHUNK + LAG_O):
        if t < NCHUNK:
            b = t % NBUF
            if t >= NBUF:
                outs[t - NBUF].wait()
            fills[t] = pltpu.async_copy(
                pe_s, bufs.at[b].at[:, pl.ds(0, EMBED_DIM)], sems[b])
        j = t - LAG_G
        if 0 <= j < NCHUNK:
            b = j % NBUF
            fills[j].wait()
            base = j * SEQ_LEN
            g1s[j] = pltpu.async_copy(
                table_hbm.at[idx_v.at[pl.ds(base, 128)]],
                bufs.at[b].at[pl.ds(0, 128)], sems[b], add=True)
            g2s[j] = pltpu.async_copy(
                table_hbm.at[idx_v.at[pl.ds(base + 128, 72)]],
                bufs.at[b].at[pl.ds(128, 72)], sems[b], add=True)
        j = t - LAG_O
        if 0 <= j < NCHUNK:
            b = j % NBUF
            g1s[j].wait()
            g2s[j].wait()
            outs[j] = pltpu.async_copy(
                bufs.at[b].at[:, pl.ds(0, EMBED_DIM)],
                out_hbm.at[wid * NCHUNK + j], sems[b])
    for j in range(NCHUNK - NBUF, NCHUNK):
        outs[j].wait()


TC_BLOCK = 2048


def _repack_body(tt_ref, out_ref):
    blk = tt_ref[...]
    out_ref[...] = jnp.concatenate(
        [blk.T, jnp.zeros((TC_BLOCK, EMBED_DIM), jnp.float32)], axis=1)


def _repack(table_t):
    grid = (VOCAB + TC_BLOCK - 1) // TC_BLOCK
    return pl.pallas_call(
        _repack_body,
        grid=(grid,),
        in_specs=[pl.BlockSpec((EMBED_DIM, TC_BLOCK), lambda i: (0, i))],
        out_specs=pl.BlockSpec((TC_BLOCK, PAD_DIM), lambda i: (i, 0)),
        out_shape=jax.ShapeDtypeStruct((VOCAB, PAD_DIM), jnp.float32),
    )(table_t)


@jax.jit
def _embed(inputs_flat, table, pe):
    mesh = plsc.VectorSubcoreMesh(core_axis_name="c", subcore_axis_name="s")
    call = functools.partial(
        pl.kernel,
        mesh=mesh,
        out_type=jax.ShapeDtypeStruct((BATCH, SEQ_LEN, EMBED_DIM), jnp.float32),
        scratch_types=[
            pltpu.VMEM((ROWS_PER_WORKER,), jnp.int32),
            pltpu.VMEM_SHARED((SEQ_LEN, EMBED_DIM), jnp.float32),
            pltpu.VMEM((NBUF, SEQ_LEN, PAD_DIM), jnp.float32),
        ] + [pltpu.SemaphoreType.DMA] * NBUF + [
        ],
        compiler_params=pltpu.CompilerParams(use_tc_tiling_on_sc=False),
    )(_sc_body)
    return call(table, inputs_flat, pe)


def kernel(inputs, table):
    idx = inputs.reshape(TOTAL).astype(jnp.int32)
    tpad = _repack(table.T)
    return _embed(idx, tpad, _PE)


# speedup vs baseline: 2.1001x; 1.5233x over previous
import functools

import jax
import jax.numpy as jnp
import numpy as np
from jax import lax
from jax.experimental import pallas as pl
from jax.experimental.pallas import tpu as pltpu
from jax.experimental.pallas import tpu_sc as plsc

SEQ_LEN = 200
VOCAB = 1000000
EMBED_DIM = 64
BATCH = 1024

NUM_CORES = 2
NUM_SUBCORES = 16
NUM_WORKERS = NUM_CORES * NUM_SUBCORES

TOTAL = BATCH * SEQ_LEN
ROWS_PER_WORKER = TOTAL // NUM_WORKERS
CHUNK = 128
CHUNKS_PER_WORKER = ROWS_PER_WORKER // CHUNK
IDX_ROWS = TOTAL // CHUNK


def _positional_encoding():
    pos = np.arange(SEQ_LEN, dtype=np.float64)[:, None]
    i = np.arange(EMBED_DIM, dtype=np.float64)[None, :]
    exponent = (i - np.mod(i, 2)) / EMBED_DIM
    angle = pos / np.power(10000.0, exponent)
    pe = np.where(np.mod(np.arange(EMBED_DIM), 2)[None, :] == 0,
                  np.sin(angle), np.cos(angle))
    return pe.astype(np.float32)


_PE = _positional_encoding()


NCHUNK = ROWS_PER_WORKER // SEQ_LEN
NBUF = 6
LAG_G = 2
LAG_O = 4
PAD_DIM = 2 * EMBED_DIM


def _sc_body(table_hbm, idx_hbm, pe_hbm, out_hbm, idx_v, pe_s, bufs, *sems):
    sid = lax.axis_index("s")
    wid = sid * NUM_CORES + lax.axis_index("c")
    row_base = wid * ROWS_PER_WORKER

    pltpu.sync_copy(idx_hbm.at[pl.ds(row_base, ROWS_PER_WORKER)], idx_v)

    @pl.when(sid == 0)
    def _():
        pltpu.sync_copy(pe_hbm, pe_s)

    plsc.subcore_barrier()

    fills, g1s, g2s, outs = {}, {}, {}, {}
    for t in range(NCHUNK + LAG_O):
        if t < NCHUNK:
            b = t % NBUF
            if t >= NBUF:
                outs[t - NBUF].wait()
            fills[t] = pltpu.async_copy(pe_s, bufs.at[b], sems[b])
        j = t - LAG_G
        if 0 <= j < NCHUNK:
            b = j % NBUF
            fills[j].wait()
            base = j * SEQ_LEN
            g1s[j] = pltpu.async_copy(
                table_hbm.at[idx_v.at[pl.ds(base, 128)]],
                bufs.at[b].at[pl.ds(0, 128)], sems[b], add=True)
            g2s[j] = pltpu.async_copy(
                table_hbm.at[idx_v.at[pl.ds(base + 128, 72)]],
                bufs.at[b].at[pl.ds(128, 72)], sems[b], add=True)
        j = t - LAG_O
        if 0 <= j < NCHUNK:
            b = j % NBUF
            g1s[j].wait()
            g2s[j].wait()
            outs[j] = pltpu.async_copy(
                bufs.at[b], out_hbm.at[wid * NCHUNK + j], sems[b])
    for j in range(NCHUNK - NBUF, NCHUNK):
        outs[j].wait()


TC_K = 4096
TC_GRID = (VOCAB + 2 * TC_K - 1) // (2 * TC_K)
PACK_ROWS = TC_K * TC_GRID
LIN_ROWS = 2 * PACK_ROWS


def _repack_body(a_ref, b_ref, out_ref):
    out_ref[...] = jnp.concatenate([a_ref[...].T, b_ref[...].T], axis=1)


def _repack(table_t):
    return pl.pallas_call(
        _repack_body,
        grid=(TC_GRID,),
        in_specs=[
            pl.BlockSpec((EMBED_DIM, TC_K), lambda i: (0, 2 * i)),
            pl.BlockSpec(
                (EMBED_DIM, TC_K),
                lambda i: (0, jnp.minimum(2 * i + 1, VOCAB // TC_K - 1)),
            ),
        ],
        out_specs=pl.BlockSpec((TC_K, PAD_DIM), lambda i: (i, 0)),
        out_shape=jax.ShapeDtypeStruct((PACK_ROWS, PAD_DIM), jnp.float32),
    )(table_t, table_t)


@jax.jit
def _embed(inputs_flat, table, pe):
    mesh = plsc.VectorSubcoreMesh(core_axis_name="c", subcore_axis_name="s")
    call = functools.partial(
        pl.kernel,
        mesh=mesh,
        out_type=jax.ShapeDtypeStruct((BATCH, SEQ_LEN, EMBED_DIM), jnp.float32),
        scratch_types=[
            pltpu.VMEM((ROWS_PER_WORKER,), jnp.int32),
            pltpu.VMEM_SHARED((SEQ_LEN, EMBED_DIM), jnp.float32),
            pltpu.VMEM((NBUF, SEQ_LEN, EMBED_DIM), jnp.float32),
        ] + [pltpu.SemaphoreType.DMA] * NBUF + [
        ],
        compiler_params=pltpu.CompilerParams(use_tc_tiling_on_sc=False),
    )(_sc_body)
    return call(table, inputs_flat, pe)


def kernel(inputs, table):
    v = inputs.reshape(TOTAL).astype(jnp.int32)
    idx = (v & ~(2 * TC_K - 1)) + 2 * (v & (TC_K - 1)) + ((v >> 12) & 1)
    t_lin = _repack(table.T).reshape(LIN_ROWS, EMBED_DIM)
    return _embed(idx, t_lin, _PE)


# speedup vs baseline: 2.2655x; 1.0788x over previous
import functools

import jax
import jax.numpy as jnp
import numpy as np
from jax import lax
from jax.experimental import pallas as pl
from jax.experimental.pallas import tpu as pltpu
from jax.experimental.pallas import tpu_sc as plsc

SEQ_LEN = 200
VOCAB = 1000000
EMBED_DIM = 64
BATCH = 1024

NUM_CORES = 2
NUM_SUBCORES = 16
NUM_WORKERS = NUM_CORES * NUM_SUBCORES

TOTAL = BATCH * SEQ_LEN
ROWS_PER_WORKER = TOTAL // NUM_WORKERS
CHUNK = 128
CHUNKS_PER_WORKER = ROWS_PER_WORKER // CHUNK
IDX_ROWS = TOTAL // CHUNK


def _positional_encoding():
    pos = np.arange(SEQ_LEN, dtype=np.float64)[:, None]
    i = np.arange(EMBED_DIM, dtype=np.float64)[None, :]
    exponent = (i - np.mod(i, 2)) / EMBED_DIM
    angle = pos / np.power(10000.0, exponent)
    pe = np.where(np.mod(np.arange(EMBED_DIM), 2)[None, :] == 0,
                  np.sin(angle), np.cos(angle))
    return pe.astype(np.float32)


_PE = _positional_encoding()


NCHUNK = ROWS_PER_WORKER // SEQ_LEN
NBUF = 8
LAG_G = 2
LAG_O = 4
PAD_DIM = 2 * EMBED_DIM


def _sc_body(table_hbm, idx_hbm, pe_hbm, out_hbm, idx_v, pe_s, bufs, *sems):
    sid = lax.axis_index("s")
    wid = sid * NUM_CORES + lax.axis_index("c")
    row_base = wid * ROWS_PER_WORKER

    pltpu.sync_copy(idx_hbm.at[pl.ds(row_base, ROWS_PER_WORKER)], idx_v)

    @pl.when(sid == 0)
    def _():
        pltpu.sync_copy(pe_hbm, pe_s)

    plsc.subcore_barrier()

    fills, g1s, g2s, outs = {}, {}, {}, {}
    for t in range(NCHUNK + LAG_O):
        if t < NCHUNK:
            b = t % NBUF
            if t >= NBUF:
                outs[t - NBUF].wait()
            fills[t] = pltpu.async_copy(pe_s, bufs.at[b], sems[b])
        j = t - LAG_G
        if 0 <= j < NCHUNK:
            b = j % NBUF
            fills[j].wait()
            base = j * SEQ_LEN
            g1s[j] = pltpu.async_copy(
                table_hbm.at[idx_v.at[pl.ds(base, 128)]],
                bufs.at[b].at[pl.ds(0, 128)], sems[b], add=True)
            g2s[j] = pltpu.async_copy(
                table_hbm.at[idx_v.at[pl.ds(base + 128, 72)]],
                bufs.at[b].at[pl.ds(128, 72)], sems[b], add=True)
        j = t - LAG_O
        if 0 <= j < NCHUNK:
            b = j % NBUF
            g1s[j].wait()
            g2s[j].wait()
            outs[j] = pltpu.async_copy(
                bufs.at[b], out_hbm.at[wid * NCHUNK + j], sems[b])
    for j in range(NCHUNK - NBUF, NCHUNK):
        outs[j].wait()


TC_K = 8192
TC_GRID = (VOCAB + 2 * TC_K - 1) // (2 * TC_K)
PACK_ROWS = TC_K * TC_GRID
LIN_ROWS = 2 * PACK_ROWS


def _repack_body(a_ref, b_ref, out_ref):
    out_ref[...] = jnp.concatenate([a_ref[...].T, b_ref[...].T], axis=1)


def _repack(table_t):
    return pl.pallas_call(
        _repack_body,
        grid=(TC_GRID,),
        in_specs=[
            pl.BlockSpec((EMBED_DIM, TC_K), lambda i: (0, 2 * i)),
            pl.BlockSpec(
                (EMBED_DIM, TC_K),
                lambda i: (0, jnp.minimum(2 * i + 1, VOCAB // TC_K - 1)),
            ),
        ],
        out_specs=pl.BlockSpec((TC_K, PAD_DIM), lambda i: (i, 0)),
        out_shape=jax.ShapeDtypeStruct((PACK_ROWS, PAD_DIM), jnp.float32),
    )(table_t, table_t)


@jax.jit
def _embed(inputs_flat, table, pe):
    mesh = plsc.VectorSubcoreMesh(core_axis_name="c", subcore_axis_name="s")
    call = functools.partial(
        pl.kernel,
        mesh=mesh,
        out_type=jax.ShapeDtypeStruct((BATCH, SEQ_LEN, EMBED_DIM), jnp.float32),
        scratch_types=[
            pltpu.VMEM((ROWS_PER_WORKER,), jnp.int32),
            pltpu.VMEM_SHARED((SEQ_LEN, EMBED_DIM), jnp.float32),
            pltpu.VMEM((NBUF, SEQ_LEN, EMBED_DIM), jnp.float32),
        ] + [pltpu.SemaphoreType.DMA] * NBUF + [
        ],
        compiler_params=pltpu.CompilerParams(use_tc_tiling_on_sc=False),
    )(_sc_body)
    return call(table, inputs_flat, pe)


def kernel(inputs, table):
    v = inputs.reshape(TOTAL).astype(jnp.int32)
    idx = (v & ~(2 * TC_K - 1)) + 2 * (v & (TC_K - 1)) + ((v >> 13) & 1)
    t_lin = _repack(table.T).reshape(LIN_ROWS, EMBED_DIM)
    return _embed(idx, t_lin, _PE)
